# baseline (speedup 1.0000x reference)
import jax
import jax.numpy as jnp
from jax import lax
from jax.experimental import pallas as pl
from jax.experimental.pallas import tpu as pltpu
from jax.experimental.pallas import tpu_sc as plsc

NUM_FIELDS = 26
VOCAB = 100000
EMBED_DIM = 32
BATCH = 16384

NC = 2
NS = 16
NW = NC * NS
L = 16

BB = 128
NU = NUM_FIELDS * (BATCH // BB)
QB = BATCH // 4
DT = EMBED_DIM // 8


def _body(xq_hbm, tab_hbm, out_hbm, idx_v, row_v, outq, gsem, isem, wsems):
    w = lax.axis_index("s") * NC + lax.axis_index("c")
    r = lax.shift_right_logical(w, 3)
    i = lax.rem(w, 8)

    def start_idx(f, q, s):
        pltpu.async_copy(
            xq_hbm.at[pl.ds(f * (BATCH // BB) + q * (QB // BB), QB // BB)],
            idx_v.at[s], isem,
        )

    def wait_idx(s):
        pltpu.make_async_copy(
            xq_hbm.at[pl.ds(0, QB // BB)], idx_v.at[s], isem
        ).wait()

    start_idx(0, 0, 0)

    def field(f, _):
        pltpu.async_copy(tab_hbm.at[f, w], row_v, gsem)

        for q in range(4):
            wait_idx(q % 2)

            if q < 3:
                start_idx(f, q + 1, (q + 1) % 2)
            else:

                @pl.when(f + 1 < NUM_FIELDS)
                def _():
                    start_idx(f + 1, 0, (q + 1) % 2)

            if q == 0:
                pltpu.make_async_copy(
                    tab_hbm.at[0, 0], row_v, gsem
                ).wait()

            for c in range(QB // BB):
                for bb in range(BB // L):
                    v = idx_v[q % 2, c, pl.ds(bb * L, L)]
                    outq[q % 2, c, pl.ds(bb * L, L)] = plsc.load_gather(
                        row_v, [v]
                    )

            @pl.when((f > 0) | (q >= 2))
            def _():
                pltpu.make_async_copy(
                    outq.at[q % 2], out_hbm.at[0, 0, pl.ds(0, QB // BB), 0],
                    wsems[q % 2],
                ).wait()

            pltpu.async_copy(
                outq.at[q % 2],
                out_hbm.at[f, r, pl.ds(q * (QB // BB), QB // BB), i],
                wsems[q % 2],
            )
        return 0

    lax.fori_loop(0, NUM_FIELDS, field, 0)
    for s in range(2):
        pltpu.make_async_copy(
            outq.at[s], out_hbm.at[0, 0, pl.ds(0, QB // BB), 0], wsems[s]
        ).wait()


@jax.jit
def _embed(x_cat, tables):
    xq = jnp.transpose(x_cat.astype(jnp.int32)).reshape(NU, BB)
    tab_t = jnp.transpose(tables, (0, 2, 1))
    mesh = plsc.VectorSubcoreMesh(core_axis_name="c", subcore_axis_name="s")
    f = pl.kernel(
        _body,
        out_type=jax.ShapeDtypeStruct(
            (NUM_FIELDS, DT, BATCH // BB, 8, BB), jnp.float32
        ),
        mesh=mesh,
        scratch_types=[
            pltpu.VMEM((2, QB // BB, BB), jnp.int32),
            pltpu.VMEM((VOCAB,), jnp.float32),
            pltpu.VMEM((2, QB // BB, BB), jnp.float32),
            pltpu.SemaphoreType.DMA,
            pltpu.SemaphoreType.DMA,
            [pltpu.SemaphoreType.DMA for _ in range(2)],
        ],
        compiler_params=pltpu.CompilerParams(
            use_tc_tiling_on_sc=True, needs_layout_passes=False
        ),
    )
    out5d = f(xq, tab_t)
    return jnp.transpose(out5d, (2, 4, 0, 1, 3)).reshape(
        BATCH, NUM_FIELDS, EMBED_DIM
    )


def kernel(x_cat, tables):
    return _embed(x_cat, tables)
